# baseline (speedup 1.0000x reference)
import functools

import jax
import jax.numpy as jnp
from jax import lax
from jax.experimental import pallas as pl
from jax.experimental.pallas import tpu as pltpu
from jax.experimental.pallas import tpu_sc as plsc

N_SRC = 6_400_000
N_DST = 100_000
NC = 2
NS = 16
NW = NC * NS
DPW = 3200
NDP = NW * DPW
ACC = DPW + 16
W = 16384
VPW = W // 16
UR = 8
H = 16
SLOT = H + W + 16
NEG = float("-inf")


SSTRIDES = (400000, 25000, 1563, 98, 7, 1)


def _sc_body(x_hbm, idx_hbm, inp_hbm, ym_hbm, ys_hbm,
             idxb, valb, accm, accs, inpb, pbuf, gbuf, semi, semv, semp):
  wid = lax.axis_index("c") * NS + lax.axis_index("s")
  dstbase = wid * DPW

  iota = lax.iota(jnp.int32, 16)
  t1v = jnp.broadcast_to(dstbase, (16,))
  t2v = jnp.broadcast_to(dstbase + DPW, (16,))
  nv = jnp.broadcast_to(N_SRC, (16,))
  s_lo = jnp.int32(0)
  s_hi = jnp.int32(0)
  for s in SSTRIDES:
    pc = (iota + 1) * s - 1
    pv1 = jnp.broadcast_to(s_lo, (16,)) + pc
    pv2 = jnp.broadcast_to(s_hi, (16,)) + pc
    pbuf[pl.ds(0, 16)] = jnp.minimum(pv1, N_SRC - 1)
    pbuf[pl.ds(16, 16)] = jnp.minimum(pv2, N_SRC - 1)
    pltpu.async_copy(idx_hbm.at[pbuf], gbuf, semp).wait()
    g1 = gbuf[pl.ds(0, 16)]
    g2 = gbuf[pl.ds(16, 16)]
    k1 = plsc.all_reduce_population_count((pv1 < nv) & (g1 < t1v))
    k2 = plsc.all_reduce_population_count((pv2 < nv) & (g2 < t2v))
    s_lo = s_lo + k1[0] * s
    s_hi = s_hi + k2[0] * s

  a0 = (s_lo // 8) * 8
  nwin = lax.max((s_hi - a0 + W - 1) // W, 0)

  shl1 = jnp.minimum(iota + 1, 15)
  li15 = jnp.full((16,), 15, jnp.int32)
  sh = {d: jnp.maximum(iota - d, 0) for d in (1, 2, 4, 8)}
  l15 = iota == 15
  nl15 = iota != 15
  zerov = jnp.zeros((16,), jnp.float32)
  nbasev = jnp.broadcast_to(-dstbase, (16,))
  negv = jnp.full((16,), NEG, jnp.float32)
  padv = jnp.full((16,), DPW, jnp.int32)
  sentv = jnp.full((16,), -1, jnp.int32)

  idxb[pl.ds(0, 16)] = sentv
  idxb[pl.ds(SLOT, 16)] = sentv

  def initb(k, c):
    accm[pl.ds(k * 16, 16)] = negv
    accs[pl.ds(k * 16, 16)] = zerov
    return c

  lax.fori_loop(0, ACC // 16, initb, 0)

  def dma_pair(k):
    par = lax.rem(k, 2)
    b = pl.multiple_of(jnp.minimum(a0 + k * W, N_SRC - W), 8)
    io = pl.multiple_of(par * SLOT + H, 8)
    vo = pl.multiple_of(par * W, 8)
    ci = pltpu.make_async_copy(idx_hbm.at[pl.ds(b, W)],
                               idxb.at[pl.ds(io, W)], semi.at[par])
    cv = pltpu.make_async_copy(x_hbm.at[pl.ds(b, W)],
                               valb.at[pl.ds(vo, W)], semv.at[par])
    return ci, cv

  @pl.when(nwin > 0)
  def _prime():
    ci, cv = dma_pair(0)
    ci.start()
    cv.start()

  def window(m, c):
    par = lax.rem(m, 2)
    ci, cv = dma_pair(m)
    ci.wait()
    cv.wait()

    @pl.when(m + 1 < nwin)
    def _next():
      ci2, cv2 = dma_pair(m + 1)
      ci2.start()
      cv2.start()

    braw = a0 + m * W
    ibase = par * SLOT + H
    vbase = par * W
    interior = (s_lo <= braw) & (braw + W <= s_hi) & (braw <= N_SRC - W)

    def fast(_):
      def fstep(i, c2):
        os = [ibase + i * (UR * 16) + u * 16 for u in range(UR)]
        vos = [vbase + i * (UR * 16) + u * 16 for u in range(UR)]
        ivs = [idxb[pl.ds(o, 16)] for o in os]
        vvs = [valb[pl.ds(vo, 16)] for vo in vos]
        ridxs = [ivs[u] + nbasev for u in range(UR)]
        ip1rs = [jnp.take_along_axis(ridxs[u], shl1, axis=0)
                 for u in range(UR)]
        mbs = [ridxs[u] != ip1rs[u] for u in range(UR)]
        m1s = [mbs[u] | l15 for u in range(UR)]
        cms = list(vvs)
        for d in (1, 2, 4, 8):
          for u in range(UR):
            sidx = idxb[pl.ds(os[u] - d, 16)]
            cd = ivs[u] == sidx
            sv = jnp.take_along_axis(cms[u], sh[d], axis=0)
            cms[u] = jnp.where(cd, jnp.maximum(cms[u], sv), cms[u])
        for u in range(1, UR):
          carry = jnp.take_along_axis(cms[u - 1], li15, axis=0)
          lastid = jnp.take_along_axis(ivs[u - 1], li15, axis=0)
          eq = ivs[u] == lastid
          cms[u] = jnp.where(eq, jnp.maximum(cms[u], carry), cms[u])
        css = [plsc.cumsum(vvs[u]) for u in range(UR)]
        curs = [plsc.load_gather(accm, [ridxs[u]]) for u in range(UR)]
        nms = [jnp.maximum(curs[u], cms[u]) for u in range(UR)]
        for u in range(UR):
          plsc.store_scatter(accm, [ridxs[u]], nms[u], mask=m1s[u])
        for u in range(UR):
          plsc.addupdate_scatter(accs, [ridxs[u]], css[u], mask=m1s[u])
          plsc.addupdate_scatter(accs, [ip1rs[u]], -css[u], mask=mbs[u])
        return c2

      lax.fori_loop(0, VPW // UR, fstep, 0)
      return 0

    def slow(_):
      b = jnp.minimum(braw, N_SRC - W)
      lo = jnp.maximum(s_lo, braw)
      lo_v = jnp.broadcast_to(lo, (16,))
      hi_v = jnp.broadcast_to(s_hi, (16,))
      bpos0 = jnp.broadcast_to(b, (16,)) + iota

      def vstep(i, c2):
        posv = bpos0 + jnp.broadcast_to(i * 16, (16,))
        valid = (posv >= lo_v) & (posv < hi_v)
        iv = idxb[pl.ds(ibase + i * 16, 16)]
        vv = valb[pl.ds(vbase + i * 16, 16)]
        ridx = jnp.where(valid, iv - dstbase, padv)
        vs = jnp.where(valid, vv, zerov)
        vm = jnp.where(valid, vv, negv)
        ip1 = jnp.take_along_axis(ridx, shl1, axis=0)
        mb = ridx != ip1
        m1 = mb | l15
        cm = vm
        for d in (1, 2, 4, 8):
          sidx = jnp.take_along_axis(ridx, sh[d], axis=0)
          cd = ridx == sidx
          sv = jnp.take_along_axis(cm, sh[d], axis=0)
          cm = jnp.where(cd, jnp.maximum(cm, sv), cm)
        cs = plsc.cumsum(vs)
        cur = plsc.load_gather(accm, [ridx])
        nm = jnp.maximum(cur, cm)
        plsc.store_scatter(accm, [ridx], nm, mask=m1)
        plsc.addupdate_scatter(accs, [ridx], cs, mask=m1)
        plsc.addupdate_scatter(accs, [ip1], -cs, mask=mb)
        return c2

      lax.fori_loop(0, VPW, vstep, 0)
      return 0

    lax.cond(interior, fast, slow, 0)
    return c

  lax.fori_loop(0, nwin, window, 0)

  tail = N_DST - (NW - 1) * DPW

  @pl.when(wid < NW - 1)
  def _load_inp():
    pltpu.sync_copy(inp_hbm.at[pl.ds(dstbase, DPW)], inpb)

  @pl.when(wid == NW - 1)
  def _load_inp_tail():
    pltpu.sync_copy(inp_hbm.at[pl.ds((NW - 1) * DPW, tail)],
                    inpb.at[pl.ds(0, tail)])

  def outb(k, c):
    s = pl.ds(k * 16, 16)
    iv = inpb[s]
    accm[s] = jnp.maximum(accm[s], iv)
    accs[s] = accs[s] + iv
    return c

  lax.fori_loop(0, DPW // 16, outb, 0)

  @pl.when(wid < NW - 1)
  def _store():
    pltpu.sync_copy(accm.at[pl.ds(0, DPW)], ym_hbm.at[pl.ds(dstbase, DPW)])
    pltpu.sync_copy(accs.at[pl.ds(0, DPW)], ys_hbm.at[pl.ds(dstbase, DPW)])

  @pl.when(wid == NW - 1)
  def _store_tail():
    pltpu.sync_copy(accm.at[pl.ds(0, tail)],
                    ym_hbm.at[pl.ds((NW - 1) * DPW, tail)])
    pltpu.sync_copy(accs.at[pl.ds(0, tail)],
                    ys_hbm.at[pl.ds((NW - 1) * DPW, tail)])


@jax.jit
def kernel(x, index, input):
  idx = index.astype(jnp.int32)

  mesh = plsc.VectorSubcoreMesh(core_axis_name="c", subcore_axis_name="s")
  run = pl.kernel(
      _sc_body,
      out_type=(
          jax.ShapeDtypeStruct((N_DST,), jnp.float32),
          jax.ShapeDtypeStruct((N_DST,), jnp.float32),
      ),
      mesh=mesh,
      scratch_types=[
          pltpu.VMEM((2 * SLOT,), jnp.int32),
          pltpu.VMEM((2 * W,), jnp.float32),
          pltpu.VMEM((ACC,), jnp.float32),
          pltpu.VMEM((ACC,), jnp.float32),
          pltpu.VMEM((DPW,), jnp.float32),
          pltpu.VMEM((32,), jnp.int32),
          pltpu.VMEM((32,), jnp.int32),
          pltpu.SemaphoreType.DMA((2,)),
          pltpu.SemaphoreType.DMA((2,)),
          pltpu.SemaphoreType.DMA,
      ],
      compiler_params=pltpu.CompilerParams(needs_layout_passes=False),
  )
  return run(x, idx, input)


# speedup vs baseline: 1.0605x; 1.0605x over previous
import functools

import jax
import jax.numpy as jnp
from jax import lax
from jax.experimental import pallas as pl
from jax.experimental.pallas import tpu as pltpu
from jax.experimental.pallas import tpu_sc as plsc

N_SRC = 6_400_000
N_DST = 100_000
NC = 2
NS = 16
NW = NC * NS
DPW = 3200
NDP = NW * DPW
ACC = DPW + 16
W = 8192
VPW = W // 16
UR = 8
H = 16
SLOT = H + W + 16
NEG = float("-inf")


SSTRIDES = (400000, 25000, 1563, 98, 7, 1)


def _sc_body(x_hbm, idx_hbm, inp_hbm, ym_hbm, ys_hbm,
             idxb, valb, accm, accs, inpb, pbuf, gbuf, semi, semv, semp):
  wid = lax.axis_index("c") * NS + lax.axis_index("s")
  dstbase = wid * DPW

  iota = lax.iota(jnp.int32, 16)
  t1v = jnp.broadcast_to(dstbase, (16,))
  t2v = jnp.broadcast_to(dstbase + DPW, (16,))
  nv = jnp.broadcast_to(N_SRC, (16,))
  s_lo = jnp.int32(0)
  s_hi = jnp.int32(0)
  for s in SSTRIDES:
    pc = (iota + 1) * s - 1
    pv1 = jnp.broadcast_to(s_lo, (16,)) + pc
    pv2 = jnp.broadcast_to(s_hi, (16,)) + pc
    pbuf[pl.ds(0, 16)] = jnp.minimum(pv1, N_SRC - 1)
    pbuf[pl.ds(16, 16)] = jnp.minimum(pv2, N_SRC - 1)
    pltpu.async_copy(idx_hbm.at[pbuf], gbuf, semp).wait()
    g1 = gbuf[pl.ds(0, 16)]
    g2 = gbuf[pl.ds(16, 16)]
    k1 = plsc.all_reduce_population_count((pv1 < nv) & (g1 < t1v))
    k2 = plsc.all_reduce_population_count((pv2 < nv) & (g2 < t2v))
    s_lo = s_lo + k1[0] * s
    s_hi = s_hi + k2[0] * s

  a0 = (s_lo // 8) * 8
  nwin = lax.max((s_hi - a0 + W - 1) // W, 0)

  shl1 = jnp.minimum(iota + 1, 15)
  li15 = jnp.full((16,), 15, jnp.int32)
  sh = {d: jnp.maximum(iota - d, 0) for d in (1, 2, 4, 8)}
  l15 = iota == 15
  nl15 = iota != 15
  zerov = jnp.zeros((16,), jnp.float32)
  nbasev = jnp.broadcast_to(-dstbase, (16,))
  negv = jnp.full((16,), NEG, jnp.float32)
  padv = jnp.full((16,), DPW, jnp.int32)
  sentv = jnp.full((16,), -1, jnp.int32)

  idxb[pl.ds(0, 16)] = sentv
  idxb[pl.ds(SLOT, 16)] = sentv

  def initb(k, c):
    accm[pl.ds(k * 16, 16)] = negv
    accs[pl.ds(k * 16, 16)] = zerov
    return c

  lax.fori_loop(0, ACC // 16, initb, 0)

  def dma_pair(k):
    par = lax.rem(k, 2)
    b = pl.multiple_of(jnp.minimum(a0 + k * W, N_SRC - W), 8)
    io = pl.multiple_of(par * SLOT + H, 8)
    vo = pl.multiple_of(par * W, 8)
    ci = pltpu.make_async_copy(idx_hbm.at[pl.ds(b, W)],
                               idxb.at[pl.ds(io, W)], semi.at[par])
    cv = pltpu.make_async_copy(x_hbm.at[pl.ds(b, W)],
                               valb.at[pl.ds(vo, W)], semv.at[par])
    return ci, cv

  @pl.when(nwin > 0)
  def _prime():
    ci, cv = dma_pair(0)
    ci.start()
    cv.start()

  def window(m, c):
    par = lax.rem(m, 2)
    ci, cv = dma_pair(m)
    ci.wait()
    cv.wait()

    @pl.when(m + 1 < nwin)
    def _next():
      ci2, cv2 = dma_pair(m + 1)
      ci2.start()
      cv2.start()

    braw = a0 + m * W
    ibase = par * SLOT + H
    vbase = par * W
    interior = (s_lo <= braw) & (braw + W <= s_hi) & (braw <= N_SRC - W)

    def fast(_):
      def fstep(i, c2):
        os = [ibase + i * (UR * 16) + u * 16 for u in range(UR)]
        vos = [vbase + i * (UR * 16) + u * 16 for u in range(UR)]
        ivs = [idxb[pl.ds(o, 16)] for o in os]
        vvs = [valb[pl.ds(vo, 16)] for vo in vos]
        ridxs = [ivs[u] + nbasev for u in range(UR)]
        ip1rs = [jnp.take_along_axis(ridxs[u], shl1, axis=0)
                 for u in range(UR)]
        mbs = [ridxs[u] != ip1rs[u] for u in range(UR)]
        m1s = [mbs[u] | l15 for u in range(UR)]
        cms = list(vvs)
        for d in (1, 2, 4, 8):
          for u in range(UR):
            sidx = idxb[pl.ds(os[u] - d, 16)]
            cd = ivs[u] == sidx
            sv = jnp.take_along_axis(cms[u], sh[d], axis=0)
            cms[u] = jnp.where(cd, jnp.maximum(cms[u], sv), cms[u])
        for u in range(1, UR):
          carry = jnp.take_along_axis(cms[u - 1], li15, axis=0)
          lastid = jnp.take_along_axis(ivs[u - 1], li15, axis=0)
          eq = ivs[u] == lastid
          cms[u] = jnp.where(eq, jnp.maximum(cms[u], carry), cms[u])
        css = [plsc.cumsum(vvs[u]) for u in range(UR)]
        curs = [plsc.load_gather(accm, [ridxs[u]]) for u in range(UR)]
        nms = [jnp.maximum(curs[u], cms[u]) for u in range(UR)]
        for u in range(UR):
          plsc.store_scatter(accm, [ridxs[u]], nms[u], mask=m1s[u])
        for u in range(UR):
          plsc.addupdate_scatter(accs, [ridxs[u]], css[u], mask=m1s[u])
          plsc.addupdate_scatter(accs, [ip1rs[u]], -css[u], mask=mbs[u])
        return c2

      lax.fori_loop(0, VPW // UR, fstep, 0)
      return 0

    def slow(_):
      b = jnp.minimum(braw, N_SRC - W)
      lo = jnp.maximum(s_lo, braw)
      lo_v = jnp.broadcast_to(lo, (16,))
      hi_v = jnp.broadcast_to(s_hi, (16,))
      bpos0 = jnp.broadcast_to(b, (16,)) + iota

      def vstep(i, c2):
        posv = bpos0 + jnp.broadcast_to(i * 16, (16,))
        valid = (posv >= lo_v) & (posv < hi_v)
        iv = idxb[pl.ds(ibase + i * 16, 16)]
        vv = valb[pl.ds(vbase + i * 16, 16)]
        ridx = jnp.where(valid, iv - dstbase, padv)
        vs = jnp.where(valid, vv, zerov)
        vm = jnp.where(valid, vv, negv)
        ip1 = jnp.take_along_axis(ridx, shl1, axis=0)
        mb = ridx != ip1
        m1 = mb | l15
        cm = vm
        for d in (1, 2, 4, 8):
          sidx = jnp.take_along_axis(ridx, sh[d], axis=0)
          cd = ridx == sidx
          sv = jnp.take_along_axis(cm, sh[d], axis=0)
          cm = jnp.where(cd, jnp.maximum(cm, sv), cm)
        cs = plsc.cumsum(vs)
        cur = plsc.load_gather(accm, [ridx])
        nm = jnp.maximum(cur, cm)
        plsc.store_scatter(accm, [ridx], nm, mask=m1)
        plsc.addupdate_scatter(accs, [ridx], cs, mask=m1)
        plsc.addupdate_scatter(accs, [ip1], -cs, mask=mb)
        return c2

      lax.fori_loop(0, VPW, vstep, 0)
      return 0

    lax.cond(interior, fast, slow, 0)
    return c

  lax.fori_loop(0, nwin, window, 0)

  tail = N_DST - (NW - 1) * DPW

  @pl.when(wid < NW - 1)
  def _load_inp():
    pltpu.sync_copy(inp_hbm.at[pl.ds(dstbase, DPW)], inpb)

  @pl.when(wid == NW - 1)
  def _load_inp_tail():
    pltpu.sync_copy(inp_hbm.at[pl.ds((NW - 1) * DPW, tail)],
                    inpb.at[pl.ds(0, tail)])

  def outb(k, c):
    s = pl.ds(k * 16, 16)
    iv = inpb[s]
    accm[s] = jnp.maximum(accm[s], iv)
    accs[s] = accs[s] + iv
    return c

  lax.fori_loop(0, DPW // 16, outb, 0)

  @pl.when(wid < NW - 1)
  def _store():
    pltpu.sync_copy(accm.at[pl.ds(0, DPW)], ym_hbm.at[pl.ds(dstbase, DPW)])
    pltpu.sync_copy(accs.at[pl.ds(0, DPW)], ys_hbm.at[pl.ds(dstbase, DPW)])

  @pl.when(wid == NW - 1)
  def _store_tail():
    pltpu.sync_copy(accm.at[pl.ds(0, tail)],
                    ym_hbm.at[pl.ds((NW - 1) * DPW, tail)])
    pltpu.sync_copy(accs.at[pl.ds(0, tail)],
                    ys_hbm.at[pl.ds((NW - 1) * DPW, tail)])


@jax.jit
def kernel(x, index, input):
  idx = index.astype(jnp.int32)

  mesh = plsc.VectorSubcoreMesh(core_axis_name="c", subcore_axis_name="s")
  run = pl.kernel(
      _sc_body,
      out_type=(
          jax.ShapeDtypeStruct((N_DST,), jnp.float32),
          jax.ShapeDtypeStruct((N_DST,), jnp.float32),
      ),
      mesh=mesh,
      scratch_types=[
          pltpu.VMEM((2 * SLOT,), jnp.int32),
          pltpu.VMEM((2 * W,), jnp.float32),
          pltpu.VMEM((ACC,), jnp.float32),
          pltpu.VMEM((ACC,), jnp.float32),
          pltpu.VMEM((DPW,), jnp.float32),
          pltpu.VMEM((32,), jnp.int32),
          pltpu.VMEM((32,), jnp.int32),
          pltpu.SemaphoreType.DMA((2,)),
          pltpu.SemaphoreType.DMA((2,)),
          pltpu.SemaphoreType.DMA,
      ],
      compiler_params=pltpu.CompilerParams(needs_layout_passes=False),
  )
  return run(x, idx, input)
